# baseline (speedup 1.0000x reference)
import functools

import jax
import jax.numpy as jnp
from jax import lax
from jax.experimental import pallas as pl
from jax.experimental.pallas import tpu as pltpu
from jax.experimental.pallas import tpu_sc as plsc

D_MODEL = 256
B = 16
T = 2048

N = B * T
NW = 32
TW = T // NW
SW = 128
NSTEP = B * TW // SW
LANES = 16
NVEC = D_MODEL // LANES
NBUF = 3
DEPTH = 2

_mesh = plsc.VectorSubcoreMesh(core_axis_name="c", subcore_axis_name="s")


@functools.partial(
    pl.kernel,
    mesh=_mesh,
    out_type=jax.ShapeDtypeStruct((N, D_MODEL), jnp.float32),
    scratch_types=[
        pltpu.VMEM((B * TW,), jnp.int32),
        pltpu.VMEM((TW, D_MODEL), jnp.float32),
    ]
    + [pltpu.VMEM((SW, D_MODEL), jnp.float32) for _ in range(NBUF)]
    + [pltpu.SemaphoreType.DMA for _ in range(2 * NBUF + 1)],
)
def _emb_lookup(x_hbm, tok_hbm, pos_hbm, out_hbm, idx_v, pos_v, *rest):
    bufs = list(rest[:NBUF])
    gsems = list(rest[NBUF : 2 * NBUF])
    osems = list(rest[2 * NBUF : 3 * NBUF])
    ssem = rest[3 * NBUF]

    wid = lax.axis_index("s") * 2 + lax.axis_index("c")
    t0 = wid * TW

    staged = [
        pltpu.async_copy(
            x_hbm.at[pl.ds(b * T + t0, TW)], idx_v.at[pl.ds(b * TW, TW)], ssem
        )
        for b in range(B)
    ]
    staged.append(pltpu.async_copy(pos_hbm.at[pl.ds(t0, TW)], pos_v, ssem))
    for d in staged:
        d.wait()

    def gather(step):
        s = step % NBUF
        return pltpu.async_copy(
            tok_hbm.at[idx_v.at[pl.ds(step * SW, SW)]], bufs[s], gsems[s]
        )

    gd = {}
    od = {}
    for step in range(DEPTH):
        gd[step % NBUF] = gather(step)

    for step in range(NSTEP):
        s = step % NBUF
        gd.pop(s).wait()

        buf = bufs[s]

        def add_row(r, carry):
            for j in range(NVEC):
                sl = pl.ds(j * LANES, LANES)
                plsc.addupdate(buf.at[r, sl], pos_v[r, sl])
                plsc.addupdate(buf.at[TW + r, sl], pos_v[r, sl])
            return carry

        lax.fori_loop(0, TW, add_row, 0)

        b0 = 2 * step
        od[s] = [
            pltpu.async_copy(
                buf.at[pl.ds(0, TW)], out_hbm.at[pl.ds(b0 * T + t0, TW)], osems[s]
            ),
            pltpu.async_copy(
                buf.at[pl.ds(TW, TW)],
                out_hbm.at[pl.ds((b0 + 1) * T + t0, TW)],
                osems[s],
            ),
        ]

        nstep = step + DEPTH
        if nstep < NSTEP:
            ns = nstep % NBUF
            if ns in od:
                for d in od.pop(ns):
                    d.wait()
            gd[ns] = gather(nstep)

    for s in sorted(od):
        for d in od.pop(s):
            d.wait()


def kernel(x, token_table, pos_table):
    xf = x.reshape(-1).astype(jnp.int32)
    out = _emb_lookup(xf, token_table, pos_table)
    return out.reshape(B, T, D_MODEL)


# speedup vs baseline: 1.2643x; 1.2643x over previous
import functools

import jax
import jax.numpy as jnp
from jax import lax
from jax.experimental import pallas as pl
from jax.experimental.pallas import tpu as pltpu
from jax.experimental.pallas import tpu_sc as plsc

D_MODEL = 256
B = 16
T = 2048

N = B * T
NW = 32
TW = T // NW
LANES = 16
NVEC = D_MODEL // LANES
SW = 32
NSTEP = 2 * B
NBUF = 8
DEPTH = 6

_mesh = plsc.VectorSubcoreMesh(core_axis_name="c", subcore_axis_name="s")


@functools.partial(
    pl.kernel,
    mesh=_mesh,
    out_type=jax.ShapeDtypeStruct((N, D_MODEL), jnp.float32),
    scratch_types=[
        pltpu.VMEM((B, 2 * TW), jnp.int32),
        pltpu.VMEM((TW, D_MODEL), jnp.float32),
    ]
    + [pltpu.VMEM((SW, D_MODEL), jnp.float32) for _ in range(NBUF)]
    + [pltpu.SemaphoreType.DMA for _ in range(2 * NBUF + 1)],
)
def _emb_lookup(x_hbm, tok_hbm, pos_hbm, out_hbm, idx_v, pos_v, *rest):
    bufs = list(rest[:NBUF])
    gsems = list(rest[NBUF : 2 * NBUF])
    osems = list(rest[2 * NBUF : 3 * NBUF])
    ssem = rest[3 * NBUF]

    wid = lax.axis_index("s") * 2 + lax.axis_index("c")
    t0 = wid * TW
    ta = pl.multiple_of((wid // 2) * (2 * TW), 2 * TW)
    off = pl.multiple_of((wid % 2) * TW, TW)

    staged = [
        pltpu.async_copy(x_hbm.at[:, pl.ds(ta, 2 * TW)], idx_v, ssem),
        pltpu.async_copy(pos_hbm.at[pl.ds(t0, TW)], pos_v, ssem),
    ]
    for d in staged:
        d.wait()

    def gather(step):
        s = step % NBUF
        b, half = step // 2, step % 2
        return pltpu.async_copy(
            tok_hbm.at[idx_v.at[b, pl.ds(off + half * SW, SW)]], bufs[s], gsems[s]
        )

    gd = {}
    od = {}
    for step in range(DEPTH):
        gd[step % NBUF] = gather(step)

    for step in range(NSTEP):
        s = step % NBUF
        b, half = step // 2, step % 2
        gd.pop(s).wait()

        buf = bufs[s]

        def add_row(r, carry):
            for j in range(NVEC):
                sl = pl.ds(j * LANES, LANES)
                plsc.addupdate(buf.at[r, sl], pos_v[half * SW + r, sl])
            return carry

        lax.fori_loop(0, SW, add_row, 0)

        od[s] = pltpu.async_copy(
            buf, out_hbm.at[pl.ds(b * T + t0 + half * SW, SW)], osems[s]
        )

        nstep = step + DEPTH
        if nstep < NSTEP:
            ns = nstep % NBUF
            if ns in od:
                od.pop(ns).wait()
            gd[ns] = gather(nstep)

    for s in sorted(od):
        od.pop(s).wait()


def kernel(x, token_table, pos_table):
    out = _emb_lookup(x.astype(jnp.int32), token_table, pos_table)
    return out.reshape(B, T, D_MODEL)
